# baseline (speedup 1.0000x reference)
import functools

import jax
import jax.numpy as jnp
import numpy as np
from jax import lax
from jax.experimental import pallas as pl
from jax.experimental.pallas import tpu as pltpu
from jax.experimental.pallas import tpu_sc as plsc

N_NODES = 10000
N_EDGES = 320000
EMB = 128
R_CUT = 5.0

F_TAB = 3 * EMB
F_DELTA = 4 * EMB

NC, NS = 2, 16
NW = NC * NS
SUBC = 5
EPW = N_EDGES // NW
EPT = N_EDGES // NS
NPAD = 10240
NPT = NPAD // NS

GB = 80
SB = 80
SGRP = 2
SROW = SB * SGRP


def _mlp_body(s_ref, v2_ref, w1_ref, b1_ref, w2_ref, b2_ref, out_ref):
    h = jax.nn.silu(
        jnp.dot(s_ref[...], w1_ref[...], preferred_element_type=jnp.float32)
        + b1_ref[...]
    )
    sp = jnp.dot(h, w2_ref[...], preferred_element_type=jnp.float32) + b2_ref[...]
    sp16 = lax.bitcast_convert_type(sp.astype(jnp.bfloat16), jnp.uint16)
    v16 = lax.bitcast_convert_type(
        v2_ref[...].astype(jnp.bfloat16), jnp.uint16
    )
    packed = sp16.astype(jnp.uint32) | (v16.astype(jnp.uint32) << 16)
    out_ref[...] = lax.bitcast_convert_type(packed, jnp.int32)


def _mlp_table(s, v2, W1, b1, W2, b2):
    bn = 1000
    grid = (N_NODES // bn,)
    return pl.pallas_call(
        _mlp_body,
        grid=grid,
        in_specs=[
            pl.BlockSpec((bn, EMB), lambda i: (i, 0)),
            pl.BlockSpec((bn, 3 * EMB), lambda i: (i, 0)),
            pl.BlockSpec((EMB, EMB), lambda i: (0, 0)),
            pl.BlockSpec((1, EMB), lambda i: (0, 0)),
            pl.BlockSpec((EMB, 3 * EMB), lambda i: (0, 0)),
            pl.BlockSpec((1, 3 * EMB), lambda i: (0, 0)),
        ],
        out_specs=pl.BlockSpec((bn, F_TAB), lambda i: (i, 0)),
        out_shape=jax.ShapeDtypeStruct((N_NODES, F_TAB), jnp.int32),
    )(s, v2, W1, b1, W2, b2)


NGB = EPW // SUBC // GB


def _gather_body(tab_hbm, src_hbm, out_hbm, idx_v, rows0, rows1, sem0, sem1):
    wid = lax.axis_index("s") * NC + lax.axis_index("c")
    base = wid * EPW
    pltpu.sync_copy(src_hbm.at[pl.ds(base, EPW)], idx_v)
    rows = (rows0, rows1)
    sems = (sem0, sem1)

    def body(e5, carry):
        off0 = e5 * (EPW // SUBC)
        descs = [None] * NGB
        descs[0] = pltpu.async_copy(
            tab_hbm.at[idx_v.at[pl.ds(off0, GB)]], rows[0], sems[0]
        )
        for i in range(NGB):
            if i + 1 < NGB:
                descs[i + 1] = pltpu.async_copy(
                    tab_hbm.at[idx_v.at[pl.ds(off0 + (i + 1) * GB, GB)]],
                    rows[(i + 1) % 2],
                    sems[(i + 1) % 2],
                )
            descs[i].wait()
            pltpu.sync_copy(
                rows[i % 2], out_hbm.at[pl.ds(base + off0 + i * GB, GB)]
            )
        return carry

    lax.fori_loop(0, SUBC, body, 0)


def _gather(tab, src):
    mesh = plsc.VectorSubcoreMesh(core_axis_name="c", subcore_axis_name="s")
    f = functools.partial(
        pl.kernel,
        out_type=jax.ShapeDtypeStruct((N_EDGES, F_TAB), jnp.int32),
        mesh=mesh,
        scratch_types=[
            pltpu.VMEM((EPW,), jnp.int32),
            pltpu.VMEM((GB, F_TAB), jnp.int32),
            pltpu.VMEM((GB, F_TAB), jnp.int32),
            pltpu.SemaphoreType.DMA,
            pltpu.SemaphoreType.DMA,
        ],
    )(_gather_body)
    return f(tab, src)


def _edge_body(r_ref, rn_ref, g_ref, wr_ref, br_ref, out_ref):
    r = r_ref[...]
    n_vals = (lax.broadcasted_iota(jnp.int32, (1, 20), 1) + 1).astype(jnp.float32)
    rbf = jnp.sin(r * (np.float32(np.pi / R_CUT)) * n_vals) / r
    rbfp = (
        jnp.dot(rbf, wr_ref[...], preferred_element_type=jnp.float32)
        + br_ref[...]
    )
    fcut = 0.5 * (jnp.cos(r * np.float32(np.pi / R_CUT)) + 1.0)
    rp = rbfp * fcut
    xu = lax.bitcast_convert_type(g_ref[...], jnp.uint32)
    sp = lax.bitcast_convert_type(
        (xu & 0xFFFF).astype(jnp.uint16), jnp.bfloat16
    ).astype(jnp.float32)
    vv = lax.bitcast_convert_type(
        (xu >> 16).astype(jnp.uint16), jnp.bfloat16
    ).astype(jnp.float32)
    po = rp * sp
    dv_gate = po[:, :EMB]
    out_ref[:, :EMB] = po[:, EMB : 2 * EMB]
    dr = po[:, 2 * EMB : 3 * EMB]
    for k in range(3):
        vk = vv[:, k * EMB : (k + 1) * EMB]
        out_ref[:, (k + 1) * EMB : (k + 2) * EMB] = (
            vk * dv_gate + rn_ref[:, k : k + 1] * dr
        )


def _edge_delta(r_ij, rn, G, W_rbf, b_rbf):
    be = 2000
    grid = (N_EDGES // be,)
    return pl.pallas_call(
        _edge_body,
        grid=grid,
        in_specs=[
            pl.BlockSpec((be, 1), lambda i: (i, 0)),
            pl.BlockSpec((be, 3), lambda i: (i, 0)),
            pl.BlockSpec((be, F_TAB), lambda i: (i, 0)),
            pl.BlockSpec((20, 3 * EMB), lambda i: (0, 0)),
            pl.BlockSpec((1, 3 * EMB), lambda i: (0, 0)),
        ],
        out_specs=pl.BlockSpec((be, F_DELTA), lambda i: (i, 0)),
        out_shape=jax.ShapeDtypeStruct((N_EDGES, F_DELTA), jnp.float32),
    )(r_ij, rn, G, W_rbf, b_rbf)


NSB = EPT // SUBC // SROW


def _scatter_body(delta_hbm, dstr_hbm, base_hbm, out_hbm, acc, idx_v,
                  dat0, dat1, sem0, sem1):
    c = lax.axis_index("c")
    sid = lax.axis_index("s")
    dats = (dat0, dat1)
    sems = (sem0, sem1)

    def do_chunk(j, carry):
        col = (c * 2 + j) * EMB
        pltpu.sync_copy(
            base_hbm.at[pl.ds(sid * NPT, NPT), pl.ds(col, EMB)],
            acc.at[pl.ds(sid * NPT, NPT)],
        )
        plsc.subcore_barrier()

        for ec in range(SUBC):
            pltpu.sync_copy(dstr_hbm.at[sid, ec], idx_v)
            row0 = sid * EPT + ec * (EPT // SUBC)
            descs = [None] * NSB
            descs[0] = pltpu.async_copy(
                delta_hbm.at[pl.ds(row0, SROW), pl.ds(col, EMB)],
                dats[0], sems[0],
            )
            for i in range(NSB):
                if i + 1 < NSB:
                    descs[i + 1] = pltpu.async_copy(
                        delta_hbm.at[
                            pl.ds(row0 + (i + 1) * SROW, SROW),
                            pl.ds(col, EMB),
                        ],
                        dats[(i + 1) % 2], sems[(i + 1) % 2],
                    )
                descs[i].wait()
                for g in range(SGRP):
                    pltpu.sync_copy(
                        dats[i % 2].at[pl.ds(g * SB, SB)],
                        acc.at[idx_v.at[SGRP * i + g, 0]], add=True,
                    )
        plsc.subcore_barrier()
        for p in range(NPT // SROW):
            b = dats[p % 2]
            pltpu.sync_copy(acc.at[pl.ds(sid * NPT + p * SROW, SROW)], b)
            pltpu.sync_copy(
                b,
                out_hbm.at[
                    pl.ds(sid * NPT + p * SROW, SROW), pl.ds(col, EMB)
                ],
            )
        plsc.subcore_barrier()
        return carry

    lax.fori_loop(0, 2, do_chunk, 0)


def _scatter(delta, dst, base_pad):
    dst_r = dst.reshape(NS, SUBC, EPT // SUBC // SB, 1, SB)
    mesh = plsc.VectorSubcoreMesh(core_axis_name="c", subcore_axis_name="s")
    f = functools.partial(
        pl.kernel,
        out_type=jax.ShapeDtypeStruct((NPAD, F_DELTA), jnp.float32),
        mesh=mesh,
        scratch_types=[
            pltpu.VMEM_SHARED((NPAD, EMB), jnp.float32),
            pltpu.VMEM((EPT // SUBC // SB, 1, SB), jnp.int32),
            pltpu.VMEM((SROW, EMB), jnp.float32),
            pltpu.VMEM((SROW, EMB), jnp.float32),
            pltpu.SemaphoreType.DMA,
            pltpu.SemaphoreType.DMA,
        ],
    )(_scatter_body)
    return f(delta, dst_r, base_pad)


def kernel(s, v, edges, r_ij, r_ij_normalized, W1, b1, W2, b2, W_rbf, b_rbf):
    v2 = v.reshape(N_NODES, 3 * EMB)
    src = edges[:, 1].astype(jnp.int32)
    dst = edges[:, 0].astype(jnp.int32)

    tab = _mlp_table(s, v2, W1, b1.reshape(1, EMB), W2, b2.reshape(1, 3 * EMB))
    base = jnp.concatenate([s, v2], axis=1)
    base_pad = jnp.pad(base, ((0, NPAD - N_NODES), (0, 0)))
    G = _gather(tab, src)
    delta = _edge_delta(r_ij, r_ij_normalized, G, W_rbf,
                        b_rbf.reshape(1, 3 * EMB))
    outc = _scatter(delta, dst, base_pad)[:N_NODES]
    return outc[:, :EMB], outc[:, EMB:].reshape(N_NODES, 3, EMB)


# speedup vs baseline: 1.0307x; 1.0307x over previous
import functools

import jax
import jax.numpy as jnp
import numpy as np
from jax import lax
from jax.experimental import pallas as pl
from jax.experimental.pallas import tpu as pltpu
from jax.experimental.pallas import tpu_sc as plsc

N_NODES = 10000
N_EDGES = 320000
EMB = 128
R_CUT = 5.0

F_TAB = 3 * EMB
F_DELTA = 4 * EMB

NC, NS = 2, 16
NW = NC * NS
NCHUNK = 5
CE = N_EDGES // NCHUNK
EPW = CE // NW
EPT = CE // NS
NPAD = 10240
NPT = NPAD // NS

GB = 80
SB = 80
SGRP = 2
SROW = SB * SGRP


def _mlp_body(s_ref, v2_ref, w1_ref, b1_ref, w2_ref, b2_ref, out_ref):
    h = jax.nn.silu(
        jnp.dot(s_ref[...], w1_ref[...], preferred_element_type=jnp.float32)
        + b1_ref[...]
    )
    sp = jnp.dot(h, w2_ref[...], preferred_element_type=jnp.float32) + b2_ref[...]
    sp16 = lax.bitcast_convert_type(sp.astype(jnp.bfloat16), jnp.uint16)
    v16 = lax.bitcast_convert_type(
        v2_ref[...].astype(jnp.bfloat16), jnp.uint16
    )
    packed = sp16.astype(jnp.uint32) | (v16.astype(jnp.uint32) << 16)
    out_ref[...] = lax.bitcast_convert_type(packed, jnp.int32)


def _mlp_table(s, v2, W1, b1, W2, b2):
    bn = 1000
    grid = (N_NODES // bn,)
    return pl.pallas_call(
        _mlp_body,
        grid=grid,
        in_specs=[
            pl.BlockSpec((bn, EMB), lambda i: (i, 0)),
            pl.BlockSpec((bn, 3 * EMB), lambda i: (i, 0)),
            pl.BlockSpec((EMB, EMB), lambda i: (0, 0)),
            pl.BlockSpec((1, EMB), lambda i: (0, 0)),
            pl.BlockSpec((EMB, 3 * EMB), lambda i: (0, 0)),
            pl.BlockSpec((1, 3 * EMB), lambda i: (0, 0)),
        ],
        out_specs=pl.BlockSpec((bn, F_TAB), lambda i: (i, 0)),
        out_shape=jax.ShapeDtypeStruct((N_NODES, F_TAB), jnp.int32),
    )(s, v2, W1, b1, W2, b2)


NGB = EPW // GB


def _gather_body(tab_hbm, src_hbm, out_hbm, idx_v, rows0, rows1, sem0, sem1):
    wid = lax.axis_index("s") * NC + lax.axis_index("c")
    base = wid * EPW
    pltpu.sync_copy(src_hbm.at[pl.ds(base, EPW)], idx_v)
    rows = (rows0, rows1)
    sems = (sem0, sem1)
    descs = [None] * NGB
    descs[0] = pltpu.async_copy(
        tab_hbm.at[idx_v.at[pl.ds(0, GB)]], rows[0], sems[0]
    )
    for i in range(NGB):
        if i + 1 < NGB:
            descs[i + 1] = pltpu.async_copy(
                tab_hbm.at[idx_v.at[pl.ds((i + 1) * GB, GB)]],
                rows[(i + 1) % 2],
                sems[(i + 1) % 2],
            )
        descs[i].wait()
        pltpu.sync_copy(rows[i % 2], out_hbm.at[pl.ds(base + i * GB, GB)])


def _gather(tab, src):
    mesh = plsc.VectorSubcoreMesh(core_axis_name="c", subcore_axis_name="s")
    f = functools.partial(
        pl.kernel,
        out_type=jax.ShapeDtypeStruct((CE, F_TAB), jnp.int32),
        mesh=mesh,
        scratch_types=[
            pltpu.VMEM((EPW,), jnp.int32),
            pltpu.VMEM((GB, F_TAB), jnp.int32),
            pltpu.VMEM((GB, F_TAB), jnp.int32),
            pltpu.SemaphoreType.DMA,
            pltpu.SemaphoreType.DMA,
        ],
    )(_gather_body)
    return f(tab, src)


def _edge_body(r_ref, rn_ref, g_ref, wr_ref, br_ref, out_ref):
    r = r_ref[...]
    n_vals = (lax.broadcasted_iota(jnp.int32, (1, 20), 1) + 1).astype(jnp.float32)
    rbf = jnp.sin(r * (np.float32(np.pi / R_CUT)) * n_vals) / r
    rbfp = (
        jnp.dot(rbf, wr_ref[...], preferred_element_type=jnp.float32)
        + br_ref[...]
    )
    fcut = 0.5 * (jnp.cos(r * np.float32(np.pi / R_CUT)) + 1.0)
    rp = rbfp * fcut
    xu = lax.bitcast_convert_type(g_ref[...], jnp.uint32)
    sp = lax.bitcast_convert_type(
        (xu & 0xFFFF).astype(jnp.uint16), jnp.bfloat16
    ).astype(jnp.float32)
    vv = lax.bitcast_convert_type(
        (xu >> 16).astype(jnp.uint16), jnp.bfloat16
    ).astype(jnp.float32)
    po = rp * sp
    dv_gate = po[:, :EMB]
    out_ref[:, :EMB] = po[:, EMB : 2 * EMB]
    dr = po[:, 2 * EMB : 3 * EMB]
    for k in range(3):
        vk = vv[:, k * EMB : (k + 1) * EMB]
        out_ref[:, (k + 1) * EMB : (k + 2) * EMB] = (
            vk * dv_gate + rn_ref[:, k : k + 1] * dr
        )


def _edge_delta(r_ij, rn, G, W_rbf, b_rbf):
    be = 1000
    grid = (CE // be,)
    return pl.pallas_call(
        _edge_body,
        grid=grid,
        in_specs=[
            pl.BlockSpec((be, 1), lambda i: (i, 0)),
            pl.BlockSpec((be, 3), lambda i: (i, 0)),
            pl.BlockSpec((be, F_TAB), lambda i: (i, 0)),
            pl.BlockSpec((20, 3 * EMB), lambda i: (0, 0)),
            pl.BlockSpec((1, 3 * EMB), lambda i: (0, 0)),
        ],
        out_specs=pl.BlockSpec((be, F_DELTA), lambda i: (i, 0)),
        out_shape=jax.ShapeDtypeStruct((CE, F_DELTA), jnp.float32),
    )(r_ij, rn, G, W_rbf, b_rbf)


NSB = EPT // SROW


def _scatter_body(*refs):
    deltas = refs[:NCHUNK]
    (dstr_hbm, base_hbm, out_hbm, acc, idx_v, dat0, dat1,
     sem0, sem1) = refs[NCHUNK:]
    c = lax.axis_index("c")
    sid = lax.axis_index("s")
    dats = (dat0, dat1)
    sems = (sem0, sem1)

    def do_chunk(j, carry):
        col = (c * 2 + j) * EMB
        pltpu.sync_copy(
            base_hbm.at[pl.ds(sid * NPT, NPT), pl.ds(col, EMB)],
            acc.at[pl.ds(sid * NPT, NPT)],
        )
        plsc.subcore_barrier()

        for ec, delta_hbm in enumerate(deltas):
            pltpu.sync_copy(dstr_hbm.at[sid, ec], idx_v)
            row0 = sid * EPT
            descs = [None] * NSB
            descs[0] = pltpu.async_copy(
                delta_hbm.at[pl.ds(row0, SROW), pl.ds(col, EMB)],
                dats[0], sems[0],
            )
            for i in range(NSB):
                if i + 1 < NSB:
                    descs[i + 1] = pltpu.async_copy(
                        delta_hbm.at[
                            pl.ds(row0 + (i + 1) * SROW, SROW),
                            pl.ds(col, EMB),
                        ],
                        dats[(i + 1) % 2], sems[(i + 1) % 2],
                    )
                descs[i].wait()
                for g in range(SGRP):
                    pltpu.sync_copy(
                        dats[i % 2].at[pl.ds(g * SB, SB)],
                        acc.at[idx_v.at[SGRP * i + g, 0]], add=True,
                    )
        plsc.subcore_barrier()
        for p in range(NPT // SROW):
            b = dats[p % 2]
            pltpu.sync_copy(acc.at[pl.ds(sid * NPT + p * SROW, SROW)], b)
            pltpu.sync_copy(
                b,
                out_hbm.at[
                    pl.ds(sid * NPT + p * SROW, SROW), pl.ds(col, EMB)
                ],
            )
        plsc.subcore_barrier()
        return carry

    lax.fori_loop(0, 2, do_chunk, 0)


def _scatter(deltas, dst, base_pad):
    dst_r = jnp.transpose(
        dst.reshape(NCHUNK, NS, EPT // SB, 1, SB), (1, 0, 2, 3, 4)
    )
    mesh = plsc.VectorSubcoreMesh(core_axis_name="c", subcore_axis_name="s")
    f = functools.partial(
        pl.kernel,
        out_type=jax.ShapeDtypeStruct((NPAD, F_DELTA), jnp.float32),
        mesh=mesh,
        scratch_types=[
            pltpu.VMEM_SHARED((NPAD, EMB), jnp.float32),
            pltpu.VMEM((EPT // SB, 1, SB), jnp.int32),
            pltpu.VMEM((SROW, EMB), jnp.float32),
            pltpu.VMEM((SROW, EMB), jnp.float32),
            pltpu.SemaphoreType.DMA,
            pltpu.SemaphoreType.DMA,
        ],
    )(_scatter_body)
    return f(*deltas, dst_r, base_pad)


def kernel(s, v, edges, r_ij, r_ij_normalized, W1, b1, W2, b2, W_rbf, b_rbf):
    v2 = v.reshape(N_NODES, 3 * EMB)
    src = edges[:, 1].astype(jnp.int32)
    dst = edges[:, 0].astype(jnp.int32)

    tab = _mlp_table(s, v2, W1, b1.reshape(1, EMB), W2, b2.reshape(1, 3 * EMB))
    base = jnp.concatenate([s, v2], axis=1)
    base_pad = jnp.pad(base, ((0, NPAD - N_NODES), (0, 0)))
    br = b_rbf.reshape(1, 3 * EMB)
    deltas = []
    for c in range(NCHUNK):
        sl = slice(c * CE, (c + 1) * CE)
        Gc = _gather(tab, src[sl])
        deltas.append(
            _edge_delta(r_ij[sl], r_ij_normalized[sl], Gc, W_rbf, br)
        )
    outc = _scatter(deltas, dst, base_pad)[:N_NODES]
    return outc[:, :EMB], outc[:, EMB:].reshape(N_NODES, 3, EMB)


# speedup vs baseline: 1.2492x; 1.2120x over previous
import functools

import jax
import jax.numpy as jnp
import numpy as np
from jax import lax
from jax.experimental import pallas as pl
from jax.experimental.pallas import tpu as pltpu
from jax.experimental.pallas import tpu_sc as plsc

N_NODES = 10000
N_EDGES = 320000
EMB = 128
R_CUT = 5.0

F_TAB = 3 * EMB
F_DELTA = 4 * EMB

NC, NS = 2, 16
NW = NC * NS
NCHUNK = 5
CE = N_EDGES // NCHUNK
EPW = CE // NW
EPT = CE // NS
NPAD = 10240
NPT = NPAD // NS

GB = 80
SB = 80
SGRP = 2
SROW = SB * SGRP


def _mlp_body(s_ref, v2_ref, w1_ref, b1_ref, w2_ref, b2_ref, out_ref):
    h = jax.nn.silu(
        jnp.dot(s_ref[...], w1_ref[...], preferred_element_type=jnp.float32)
        + b1_ref[...]
    )
    sp = jnp.dot(h, w2_ref[...], preferred_element_type=jnp.float32) + b2_ref[...]
    sp16 = lax.bitcast_convert_type(sp.astype(jnp.bfloat16), jnp.uint16)
    v16 = lax.bitcast_convert_type(
        v2_ref[...].astype(jnp.bfloat16), jnp.uint16
    )
    packed = sp16.astype(jnp.uint32) | (v16.astype(jnp.uint32) << 16)
    out_ref[...] = lax.bitcast_convert_type(packed, jnp.int32)


def _mlp_table(s, v2, W1, b1, W2, b2):
    bn = 1000
    grid = (N_NODES // bn,)
    return pl.pallas_call(
        _mlp_body,
        grid=grid,
        in_specs=[
            pl.BlockSpec((bn, EMB), lambda i: (i, 0)),
            pl.BlockSpec((bn, 3 * EMB), lambda i: (i, 0)),
            pl.BlockSpec((EMB, EMB), lambda i: (0, 0)),
            pl.BlockSpec((1, EMB), lambda i: (0, 0)),
            pl.BlockSpec((EMB, 3 * EMB), lambda i: (0, 0)),
            pl.BlockSpec((1, 3 * EMB), lambda i: (0, 0)),
        ],
        out_specs=pl.BlockSpec((bn, F_TAB), lambda i: (i, 0)),
        out_shape=jax.ShapeDtypeStruct((N_NODES, F_TAB), jnp.int32),
    )(s, v2, W1, b1, W2, b2)


NGB = EPW // GB


def _gather_body(tab_hbm, src_hbm, out_hbm, idx_v, rows0, rows1, sem0, sem1):
    wid = lax.axis_index("s") * NC + lax.axis_index("c")
    base = wid * EPW
    pltpu.sync_copy(src_hbm.at[pl.ds(base, EPW)], idx_v)
    rows = (rows0, rows1)
    sems = (sem0, sem1)
    descs = [None] * NGB
    descs[0] = pltpu.async_copy(
        tab_hbm.at[idx_v.at[pl.ds(0, GB)]], rows[0], sems[0]
    )
    for i in range(NGB):
        if i + 1 < NGB:
            descs[i + 1] = pltpu.async_copy(
                tab_hbm.at[idx_v.at[pl.ds((i + 1) * GB, GB)]],
                rows[(i + 1) % 2],
                sems[(i + 1) % 2],
            )
        descs[i].wait()
        pltpu.sync_copy(rows[i % 2], out_hbm.at[pl.ds(base + i * GB, GB)])


def _gather(tab, src):
    mesh = plsc.VectorSubcoreMesh(core_axis_name="c", subcore_axis_name="s")
    f = functools.partial(
        pl.kernel,
        out_type=jax.ShapeDtypeStruct((CE, F_TAB), jnp.int32),
        mesh=mesh,
        scratch_types=[
            pltpu.VMEM((EPW,), jnp.int32),
            pltpu.VMEM((GB, F_TAB), jnp.int32),
            pltpu.VMEM((GB, F_TAB), jnp.int32),
            pltpu.SemaphoreType.DMA,
            pltpu.SemaphoreType.DMA,
        ],
    )(_gather_body)
    return f(tab, src)


def _edge_body(r_ref, rn_ref, g_ref, wr_ref, br_ref, out_ref):
    r = r_ref[...]
    n_vals = (lax.broadcasted_iota(jnp.int32, (1, 20), 1) + 1).astype(jnp.float32)
    smat = jnp.sin(r * (np.float32(np.pi / R_CUT)) * n_vals)
    m = jnp.dot(smat, wr_ref[...], preferred_element_type=jnp.float32)
    fcut = 0.25 * smat[:, 1:2] / smat[:, 0:1] + 0.5
    rp = m * (fcut / r) + br_ref[...] * fcut
    xu = lax.bitcast_convert_type(g_ref[...], jnp.uint32)
    sp = lax.bitcast_convert_type(
        (xu & 0xFFFF).astype(jnp.uint16), jnp.bfloat16
    ).astype(jnp.float32)
    vv = lax.bitcast_convert_type(
        (xu >> 16).astype(jnp.uint16), jnp.bfloat16
    ).astype(jnp.float32)
    po = rp * sp
    dv_gate = po[:, :EMB]
    out_ref[:, :EMB] = po[:, EMB : 2 * EMB]
    dr = po[:, 2 * EMB : 3 * EMB]
    for k in range(3):
        vk = vv[:, k * EMB : (k + 1) * EMB]
        out_ref[:, (k + 1) * EMB : (k + 2) * EMB] = (
            vk * dv_gate + rn_ref[:, k : k + 1] * dr
        )


def _edge_delta(r_ij, rn, G, W_rbf, b_rbf):
    be = 2000
    grid = (CE // be,)
    return pl.pallas_call(
        _edge_body,
        grid=grid,
        in_specs=[
            pl.BlockSpec((be, 1), lambda i: (i, 0)),
            pl.BlockSpec((be, 3), lambda i: (i, 0)),
            pl.BlockSpec((be, F_TAB), lambda i: (i, 0)),
            pl.BlockSpec((20, 3 * EMB), lambda i: (0, 0)),
            pl.BlockSpec((1, 3 * EMB), lambda i: (0, 0)),
        ],
        out_specs=pl.BlockSpec((be, F_DELTA), lambda i: (i, 0)),
        out_shape=jax.ShapeDtypeStruct((CE, F_DELTA), jnp.float32),
    )(r_ij, rn, G, W_rbf, b_rbf)


NSB = EPT // SROW


def _scatter_body(*refs):
    deltas = refs[:NCHUNK]
    (dstr_hbm, base_hbm, out_hbm, acc, idx_v, dat0, dat1,
     sem0, sem1) = refs[NCHUNK:]
    c = lax.axis_index("c")
    sid = lax.axis_index("s")
    dats = (dat0, dat1)
    sems = (sem0, sem1)

    def do_chunk(j, carry):
        col = (c * 2 + j) * EMB
        pltpu.sync_copy(
            base_hbm.at[pl.ds(sid * NPT, NPT), pl.ds(col, EMB)],
            acc.at[pl.ds(sid * NPT, NPT)],
        )
        plsc.subcore_barrier()

        for ec, delta_hbm in enumerate(deltas):
            pltpu.sync_copy(dstr_hbm.at[sid, ec], idx_v)
            row0 = sid * EPT
            descs = [None] * NSB
            descs[0] = pltpu.async_copy(
                delta_hbm.at[pl.ds(row0, SROW), pl.ds(col, EMB)],
                dats[0], sems[0],
            )
            for i in range(NSB):
                if i + 1 < NSB:
                    descs[i + 1] = pltpu.async_copy(
                        delta_hbm.at[
                            pl.ds(row0 + (i + 1) * SROW, SROW),
                            pl.ds(col, EMB),
                        ],
                        dats[(i + 1) % 2], sems[(i + 1) % 2],
                    )
                descs[i].wait()
                for g in range(SGRP):
                    pltpu.sync_copy(
                        dats[i % 2].at[pl.ds(g * SB, SB)],
                        acc.at[idx_v.at[SGRP * i + g, 0]], add=True,
                    )
        plsc.subcore_barrier()
        for p in range(NPT // SROW):
            b = dats[p % 2]
            pltpu.sync_copy(acc.at[pl.ds(sid * NPT + p * SROW, SROW)], b)
            pltpu.sync_copy(
                b,
                out_hbm.at[
                    pl.ds(sid * NPT + p * SROW, SROW), pl.ds(col, EMB)
                ],
            )
        plsc.subcore_barrier()
        return carry

    lax.fori_loop(0, 2, do_chunk, 0)


def _scatter(deltas, dst, base_pad):
    dst_r = jnp.transpose(
        dst.reshape(NCHUNK, NS, EPT // SB, 1, SB), (1, 0, 2, 3, 4)
    )
    mesh = plsc.VectorSubcoreMesh(core_axis_name="c", subcore_axis_name="s")
    f = functools.partial(
        pl.kernel,
        out_type=jax.ShapeDtypeStruct((NPAD, F_DELTA), jnp.float32),
        mesh=mesh,
        scratch_types=[
            pltpu.VMEM_SHARED((NPAD, EMB), jnp.float32),
            pltpu.VMEM((EPT // SB, 1, SB), jnp.int32),
            pltpu.VMEM((SROW, EMB), jnp.float32),
            pltpu.VMEM((SROW, EMB), jnp.float32),
            pltpu.SemaphoreType.DMA,
            pltpu.SemaphoreType.DMA,
        ],
    )(_scatter_body)
    return f(*deltas, dst_r, base_pad)


def kernel(s, v, edges, r_ij, r_ij_normalized, W1, b1, W2, b2, W_rbf, b_rbf):
    v2 = v.reshape(N_NODES, 3 * EMB)
    src = edges[:, 1].astype(jnp.int32)
    dst = edges[:, 0].astype(jnp.int32)

    tab = _mlp_table(s, v2, W1, b1.reshape(1, EMB), W2, b2.reshape(1, 3 * EMB))
    base = jnp.concatenate([s, v2], axis=1)
    base_pad = jnp.pad(base, ((0, NPAD - N_NODES), (0, 0)))
    br = b_rbf.reshape(1, 3 * EMB)
    deltas = []
    for c in range(NCHUNK):
        sl = slice(c * CE, (c + 1) * CE)
        Gc = _gather(tab, src[sl])
        deltas.append(
            _edge_delta(r_ij[sl], r_ij_normalized[sl], Gc, W_rbf, br)
        )
    outc = _scatter(deltas, dst, base_pad)[:N_NODES]
    return outc[:, :EMB], outc[:, EMB:].reshape(N_NODES, 3, EMB)
